# baseline (speedup 1.0000x reference)
import functools

import jax
import jax.numpy as jnp
from jax import lax
from jax.experimental import pallas as pl
from jax.experimental.pallas import tpu as pltpu
from jax.experimental.pallas import tpu_sc as plsc

BATCH = 16384
EMBED_DIM = 768

_INFO = plsc.get_sparse_core_info()
_NC = _INFO.num_cores
_NS = _INFO.num_subcores
_NW = _NC * _NS
_B_PER_W = BATCH // _NW
_CHUNK = 64
_N_CHUNKS = _B_PER_W // _CHUNK


def _embed_lookup(idx, table):
    mesh = plsc.VectorSubcoreMesh(core_axis_name="c", subcore_axis_name="s")

    @functools.partial(
        pl.kernel,
        mesh=mesh,
        out_type=jax.ShapeDtypeStruct((BATCH, EMBED_DIM), jnp.float32),
        scratch_types=[
            pltpu.VMEM((_B_PER_W,), jnp.int32),
            pltpu.VMEM((_CHUNK, EMBED_DIM), jnp.float32),
            pltpu.VMEM((_CHUNK, EMBED_DIM), jnp.float32),
            pltpu.SemaphoreType.DMA,
            pltpu.SemaphoreType.DMA,
            pltpu.SemaphoreType.DMA,
            pltpu.SemaphoreType.DMA,
        ],
    )
    def k(table_hbm, idx_hbm, out_hbm, idx_v, rows0, rows1, g0, g1, s0, s1):
        wid = lax.axis_index("s") * _NC + lax.axis_index("c")
        base = wid * _B_PER_W
        pltpu.sync_copy(idx_hbm.at[pl.ds(base, _B_PER_W)], idx_v)

        bufs = (rows0, rows1)
        gsems = (g0, g1)
        ssems = (s0, s1)

        def gather(i):
            return pltpu.async_copy(
                table_hbm.at[idx_v.at[pl.ds(i * _CHUNK, _CHUNK)]],
                bufs[i % 2],
                gsems[i % 2],
            )

        def scatter(i):
            return pltpu.async_copy(
                bufs[i % 2],
                out_hbm.at[pl.ds(base + i * _CHUNK, _CHUNK)],
                ssems[i % 2],
            )

        gathers = [None] * _N_CHUNKS
        scatters = [None] * _N_CHUNKS
        gathers[0] = gather(0)
        for i in range(_N_CHUNKS):
            gathers[i].wait()
            if i + 1 < _N_CHUNKS:
                if i >= 1:
                    scatters[i - 1].wait()
                gathers[i + 1] = gather(i + 1)
            scatters[i] = scatter(i)
        if _N_CHUNKS >= 2:
            scatters[_N_CHUNKS - 2].wait()
        scatters[_N_CHUNKS - 1].wait()

    return k(table, idx)


def kernel(c, text_embeddings):
    idx = c.astype(jnp.int32)
    return _embed_lookup(idx, text_embeddings)


# speedup vs baseline: 1.8983x; 1.8983x over previous
import functools

import jax
import jax.numpy as jnp
from jax import lax
from jax.experimental import pallas as pl
from jax.experimental.pallas import tpu as pltpu
from jax.experimental.pallas import tpu_sc as plsc

BATCH = 16384
EMBED_DIM = 768
NUM_CLASSES = 100

_INFO = plsc.get_sparse_core_info()
_NC = _INFO.num_cores
_NS = _INFO.num_subcores
_NW = _NC * _NS
_B_PER_W = BATCH // _NW
_WINDOW = 32


def _embed_lookup(idx, table):
    mesh = plsc.VectorSubcoreMesh(core_axis_name="c", subcore_axis_name="s")

    @functools.partial(
        pl.kernel,
        mesh=mesh,
        out_type=jax.ShapeDtypeStruct((BATCH, EMBED_DIM), jnp.float32),
        scratch_types=[
            pltpu.VMEM((NUM_CLASSES, EMBED_DIM), jnp.float32),
            pltpu.VMEM((_B_PER_W,), jnp.int32),
            pltpu.SemaphoreType.DMA,
        ],
    )
    def k(table_hbm, idx_hbm, out_hbm, table_v, idx_v, sem):
        wid = lax.axis_index("s") * _NC + lax.axis_index("c")
        base = wid * _B_PER_W
        pltpu.sync_copy(table_hbm, table_v)
        pltpu.sync_copy(idx_hbm.at[pl.ds(base, _B_PER_W)], idx_v)

        def fire_group(g):
            v16 = idx_v[pl.ds(g * 16, 16)]
            for j in range(16):
                row = v16[j]
                pltpu.async_copy(
                    table_v.at[row], out_hbm.at[base + g * 16 + j], sem
                )

        def drain_one():
            pltpu.make_async_copy(out_hbm.at[base], table_v.at[0], sem).wait()

        n_groups = _B_PER_W // 16
        w_groups = _WINDOW // 16

        def head(g, carry):
            fire_group(g)
            return carry

        def steady(g, carry):
            for _ in range(16):
                drain_one()
            fire_group(g)
            return carry

        def tail(g, carry):
            for _ in range(16):
                drain_one()
            return carry

        lax.fori_loop(0, w_groups, head, 0)
        lax.fori_loop(w_groups, n_groups, steady, 0)
        lax.fori_loop(0, w_groups, tail, 0)

    return k(table, idx)


def kernel(c, text_embeddings):
    idx = c.astype(jnp.int32)
    return _embed_lookup(idx, text_embeddings)


# speedup vs baseline: 2.1475x; 1.1313x over previous
import functools

import jax
import jax.numpy as jnp
from jax import lax
from jax.experimental import pallas as pl
from jax.experimental.pallas import tpu as pltpu
from jax.experimental.pallas import tpu_sc as plsc

BATCH = 16384
EMBED_DIM = 768
NUM_CLASSES = 100

_INFO = plsc.get_sparse_core_info()
_NC = _INFO.num_cores
_NS = _INFO.num_subcores
_NW = _NC * _NS
_B_PER_W = BATCH // _NW
_WINDOW = 32
_ALIGNED_ROWS = (NUM_CLASSES // 8) * 8
_TAIL_ROWS = NUM_CLASSES - _ALIGNED_ROWS
_N_STAGERS = _ALIGNED_ROWS // 8


def _embed_lookup(idx, table):
    mesh = plsc.VectorSubcoreMesh(core_axis_name="c", subcore_axis_name="s")

    @functools.partial(
        pl.kernel,
        mesh=mesh,
        out_type=jax.ShapeDtypeStruct((BATCH, EMBED_DIM), jnp.float32),
        scratch_types=[
            pltpu.VMEM_SHARED((NUM_CLASSES, EMBED_DIM), jnp.float32),
            pltpu.VMEM((NUM_CLASSES, EMBED_DIM), jnp.float32),
            pltpu.VMEM((_B_PER_W,), jnp.int32),
            pltpu.SemaphoreType.DMA,
        ],
    )
    def k(table_hbm, idx_hbm, out_hbm, table_sh, table_v, idx_v, sem):
        sid = lax.axis_index("s")
        wid = sid * _NC + lax.axis_index("c")
        base = wid * _B_PER_W

        @pl.when(sid < _N_STAGERS)
        def _():
            rs = pl.ds(sid * 8, 8)
            pltpu.sync_copy(table_hbm.at[rs], table_sh.at[rs])

        pltpu.sync_copy(idx_hbm.at[pl.ds(base, _B_PER_W)], idx_v)
        pltpu.sync_copy(
            table_hbm.at[pl.ds(_ALIGNED_ROWS, _TAIL_ROWS)],
            table_v.at[pl.ds(_ALIGNED_ROWS, _TAIL_ROWS)],
        )
        plsc.subcore_barrier()
        pltpu.sync_copy(
            table_sh.at[pl.ds(0, _ALIGNED_ROWS)],
            table_v.at[pl.ds(0, _ALIGNED_ROWS)],
        )

        def fire_group(g):
            v16 = idx_v[pl.ds(g * 16, 16)]
            for j in range(16):
                row = v16[j]
                pltpu.async_copy(
                    table_v.at[row], out_hbm.at[base + g * 16 + j], sem
                )

        def drain_one():
            pltpu.make_async_copy(out_hbm.at[base], table_v.at[0], sem).wait()

        n_groups = _B_PER_W // 16
        w_groups = _WINDOW // 16

        def head(g, carry):
            fire_group(g)
            return carry

        def steady(g, carry):
            for _ in range(16):
                drain_one()
            fire_group(g)
            return carry

        def tail(g, carry):
            for _ in range(16):
                drain_one()
            return carry

        lax.fori_loop(0, w_groups, head, 0)
        lax.fori_loop(w_groups, n_groups, steady, 0)
        lax.fori_loop(0, w_groups, tail, 0)

    return k(table, idx)


def kernel(c, text_embeddings):
    idx = c.astype(jnp.int32)
    return _embed_lookup(idx, text_embeddings)


# speedup vs baseline: 2.1502x; 1.0013x over previous
import functools

import jax
import jax.numpy as jnp
from jax import lax
from jax.experimental import pallas as pl
from jax.experimental.pallas import tpu as pltpu
from jax.experimental.pallas import tpu_sc as plsc

BATCH = 16384
EMBED_DIM = 768
NUM_CLASSES = 100

_INFO = plsc.get_sparse_core_info()
_NC = _INFO.num_cores
_NS = _INFO.num_subcores
_NW = _NC * _NS
_B_PER_W = BATCH // _NW
_WINDOW = 64
_ALIGNED_ROWS = (NUM_CLASSES // 8) * 8
_TAIL_ROWS = NUM_CLASSES - _ALIGNED_ROWS
_N_STAGERS = _ALIGNED_ROWS // 8


def _embed_lookup(idx, table):
    mesh = plsc.VectorSubcoreMesh(core_axis_name="c", subcore_axis_name="s")

    @functools.partial(
        pl.kernel,
        mesh=mesh,
        out_type=jax.ShapeDtypeStruct((BATCH, EMBED_DIM), jnp.float32),
        scratch_types=[
            pltpu.VMEM_SHARED((NUM_CLASSES, EMBED_DIM), jnp.float32),
            pltpu.VMEM((NUM_CLASSES, EMBED_DIM), jnp.float32),
            pltpu.VMEM((_B_PER_W,), jnp.int32),
            pltpu.SemaphoreType.DMA,
        ],
    )
    def k(table_hbm, idx_hbm, out_hbm, table_sh, table_v, idx_v, sem):
        sid = lax.axis_index("s")
        wid = sid * _NC + lax.axis_index("c")
        base = wid * _B_PER_W

        @pl.when(sid < _N_STAGERS)
        def _():
            rs = pl.ds(sid * 8, 8)
            pltpu.sync_copy(table_hbm.at[rs], table_sh.at[rs])

        pltpu.sync_copy(idx_hbm.at[pl.ds(base, _B_PER_W)], idx_v)
        pltpu.sync_copy(
            table_hbm.at[pl.ds(_ALIGNED_ROWS, _TAIL_ROWS)],
            table_v.at[pl.ds(_ALIGNED_ROWS, _TAIL_ROWS)],
        )
        plsc.subcore_barrier()
        pltpu.sync_copy(
            table_sh.at[pl.ds(0, _ALIGNED_ROWS)],
            table_v.at[pl.ds(0, _ALIGNED_ROWS)],
        )

        def fire_group(g):
            v16 = idx_v[pl.ds(g * 16, 16)]
            for j in range(16):
                row = v16[j]
                pltpu.async_copy(
                    table_v.at[row], out_hbm.at[base + g * 16 + j], sem
                )

        def drain_one():
            pltpu.make_async_copy(out_hbm.at[base], table_v.at[0], sem).wait()

        n_groups = _B_PER_W // 16
        w_groups = _WINDOW // 16

        def head(g, carry):
            fire_group(g)
            return carry

        def steady(g, carry):
            for _ in range(16):
                drain_one()
            fire_group(g)
            return carry

        def tail(g, carry):
            for _ in range(16):
                drain_one()
            return carry

        lax.fori_loop(0, w_groups, head, 0)
        lax.fori_loop(w_groups, n_groups, steady, 0)
        lax.fori_loop(0, w_groups, tail, 0)

    return k(table, idx)


def kernel(c, text_embeddings):
    idx = c.astype(jnp.int32)
    return _embed_lookup(idx, text_embeddings)


# speedup vs baseline: 2.1976x; 1.0220x over previous
import functools

import jax
import jax.numpy as jnp
from jax import lax
from jax.experimental import pallas as pl
from jax.experimental.pallas import tpu as pltpu
from jax.experimental.pallas import tpu_sc as plsc

BATCH = 16384
EMBED_DIM = 768
NUM_CLASSES = 100

_INFO = plsc.get_sparse_core_info()
_NC = _INFO.num_cores
_NS = _INFO.num_subcores
_NW = _NC * _NS
_B_PER_W = BATCH // _NW
_WINDOW = 64
_SPMEM_ROWS = 72
_HBM_ROWS = NUM_CLASSES - _SPMEM_ROWS
_N_STAGERS = _SPMEM_ROWS // 8


def _embed_lookup(idx, table):
    mesh = plsc.VectorSubcoreMesh(core_axis_name="c", subcore_axis_name="s")

    @functools.partial(
        pl.kernel,
        mesh=mesh,
        out_type=jax.ShapeDtypeStruct((BATCH, EMBED_DIM), jnp.float32),
        scratch_types=[
            pltpu.VMEM_SHARED((NUM_CLASSES, EMBED_DIM), jnp.float32),
            pltpu.VMEM((NUM_CLASSES, EMBED_DIM), jnp.float32),
            pltpu.VMEM((_B_PER_W,), jnp.int32),
            pltpu.SemaphoreType.DMA,
            pltpu.SemaphoreType.DMA,
        ],
    )
    def k(table_hbm, idx_hbm, out_hbm, table_sh, table_v, idx_v, sem, stage_sem):
        sid = lax.axis_index("s")
        wid = sid * _NC + lax.axis_index("c")
        base = wid * _B_PER_W

        @pl.when(sid < _N_STAGERS)
        def _():
            rs = pl.ds(sid * 8, 8)
            pltpu.sync_copy(table_hbm.at[rs], table_sh.at[rs])

        hbm_part = pltpu.async_copy(
            table_hbm.at[pl.ds(_SPMEM_ROWS, _HBM_ROWS)],
            table_v.at[pl.ds(_SPMEM_ROWS, _HBM_ROWS)],
            stage_sem,
        )
        pltpu.sync_copy(idx_hbm.at[pl.ds(base, _B_PER_W)], idx_v)
        plsc.subcore_barrier()
        pltpu.sync_copy(
            table_sh.at[pl.ds(0, _SPMEM_ROWS)],
            table_v.at[pl.ds(0, _SPMEM_ROWS)],
        )
        hbm_part.wait()

        def fire_group(g):
            v16 = idx_v[pl.ds(g * 16, 16)]
            for j in range(16):
                row = v16[j]
                pltpu.async_copy(
                    table_v.at[row], out_hbm.at[base + g * 16 + j], sem
                )

        def drain_one():
            pltpu.make_async_copy(out_hbm.at[base], table_v.at[0], sem).wait()

        n_groups = _B_PER_W // 16
        w_groups = _WINDOW // 16

        def head(g, carry):
            fire_group(g)
            return carry

        def steady(g, carry):
            for _ in range(16):
                drain_one()
            fire_group(g)
            return carry

        def tail(g, carry):
            for _ in range(16):
                drain_one()
            return carry

        lax.fori_loop(0, w_groups, head, 0)
        lax.fori_loop(w_groups, n_groups, steady, 0)
        lax.fori_loop(0, w_groups, tail, 0)

    return k(table, idx)


def kernel(c, text_embeddings):
    idx = c.astype(jnp.int32)
    return _embed_lookup(idx, text_embeddings)
